# baseline (speedup 1.0000x reference)
import jax
import jax.numpy as jnp
from jax import lax
from jax.experimental import pallas as pl
from jax.experimental.pallas import tpu as pltpu
from jax.experimental.pallas import tpu_sc as plsc

NC = 2
NS = 16
LANES = 16
NBUF = 3

ROWS = 320000
D = 128
SEGS = 10000
SEGS_PAD = SEGS + 240

SC_ROWS = 192000
RPC = SC_ROWS // NC
BLK = 128
NBLK = RPC // BLK
KPT = (NBLK + NS - 1) // NS

TC_ROWS = ROWS - SC_ROWS
TCR = 512
TC_NBLK = TC_ROWS // TCR
TCW = 128

NZFULL = SEGS // BLK
ZTAIL = SEGS - NZFULL * BLK


def _sc_body(feats_hbm, ids_hbm, out_hbm, bufs, idxs, acc, sem_l, sem_s):
    c = lax.axis_index("c")
    s = lax.axis_index("s")

    def fire_load(k, slot):
        r0 = (c * NBLK + s + k * NS) * BLK
        pltpu.async_copy(feats_hbm.at[pl.ds(r0, BLK), :],
                         bufs[slot], sem_l[slot])
        pltpu.async_copy(ids_hbm.at[pl.ds(r0, BLK)], idxs[slot], sem_l[slot])

    def drain_load(slot):
        pltpu.make_async_copy(feats_hbm.at[pl.ds(0, BLK), :],
                              bufs[slot], sem_l[slot]).wait()
        pltpu.make_async_copy(ids_hbm.at[pl.ds(0, BLK)],
                              idxs[slot], sem_l[slot]).wait()

    def fire_scatter(slot):
        pltpu.async_copy(bufs[slot], acc.at[idxs[slot]], sem_s, add=True)

    def drain_scatter(slot):
        pltpu.make_async_copy(bufs[slot], acc.at[idxs[slot]], sem_s).wait()

    def valid(k):
        return (s + k * NS) < NBLK

    zeros16 = jnp.zeros((LANES,), jnp.float32)

    def zero_row(i, _):
        for t in range(D // LANES):
            bufs[0][i, pl.ds(t * LANES, LANES)] = zeros16
        return 0

    lax.fori_loop(0, BLK, zero_row, 0)

    def zero_step(i, _):
        z = s + i * NS

        @pl.when(z < NZFULL)
        def _():
            pltpu.sync_copy(bufs[0], acc.at[pl.ds(z * BLK, BLK), :])

        return 0

    lax.fori_loop(0, (NZFULL + NS - 1) // NS, zero_step, 0)

    @pl.when(s == NZFULL % NS)
    def _():
        pltpu.sync_copy(bufs[0].at[pl.ds(0, ZTAIL), :],
                        acc.at[pl.ds(NZFULL * BLK, ZTAIL), :])

    plsc.subcore_barrier()

    @pl.when(valid(0))
    def _():
        fire_load(0, 0)

    @pl.when(valid(1))
    def _():
        fire_load(1, 1)

    def step(it, _):
        for r in range(NBUF):
            k = NBUF * it + r

            @pl.when(valid(k))
            def _():
                drain_load(r)
                fire_scatter(r)

                @pl.when(k >= 1)
                def _():
                    drain_scatter((r + NBUF - 1) % NBUF)

                @pl.when(valid(k + 2))
                def _():
                    fire_load(k + 2, (r + 2) % NBUF)

        return 0

    lax.fori_loop(0, (KPT + NBUF - 1) // NBUF, step, 0)

    drain_scatter(0)

    plsc.subcore_barrier()

    def drain_step(i, _):
        z = s + i * NS

        @pl.when(z < NZFULL)
        def _():
            pltpu.sync_copy(acc.at[pl.ds(z * BLK, BLK), :],
                            out_hbm.at[c, pl.ds(z * BLK, BLK), :])

        return 0

    lax.fori_loop(0, (NZFULL + NS - 1) // NS, drain_step, 0)

    @pl.when(s == NZFULL % NS)
    def _():
        pltpu.sync_copy(acc.at[pl.ds(NZFULL * BLK, ZTAIL), :],
                        out_hbm.at[c, pl.ds(NZFULL * BLK, ZTAIL), :])


def _sc_body_flat(feats_hbm, ids_hbm, out_hbm,
                  b0, b1, b2, i0, i1, i2,
                  acc, sl0, sl1, sl2, sem_s):
    _sc_body(feats_hbm, ids_hbm, out_hbm,
             (b0, b1, b2), (i0, i1, i2),
             acc, (sl0, sl1, sl2), sem_s)


def _tc_body(feats_ref, ids_ref, out_ref):
    @pl.when(pl.program_id(0) == 0)
    def _():
        out_ref[...] = jnp.zeros((SEGS_PAD, D), jnp.float32)

    ids_blk = ids_ref[0]
    rows = feats_ref[...]
    lo = jnp.min(ids_blk)
    hi = jnp.max(ids_blk)
    lo8 = (lo // 8) * 8
    nch = (hi - lo8) // TCW + 1

    def chunk(ch, _):
        base = lo8 + ch * TCW
        seg_iota = base + lax.broadcasted_iota(jnp.int32, (TCW, TCR), 0)
        oh = (seg_iota == ids_blk).astype(jnp.float32)
        part = lax.dot_general(oh, rows, (((1,), (0,)), ((), ())),
                               preferred_element_type=jnp.float32)
        out_ref[pl.ds(base, TCW), :] += part
        return 0

    lax.fori_loop(0, nch, chunk, 0)


def _combine_body(p_ref, t_ref, o_ref):
    o_ref[...] = p_ref[0] + p_ref[1] + t_ref[...]


@jax.jit
def _run(feats, segment_ids, num_segments):
    del num_segments
    ids = segment_ids.astype(jnp.int32)

    mesh = plsc.VectorSubcoreMesh(core_axis_name="c", subcore_axis_name="s")
    sc_kernel = pl.kernel(
        _sc_body_flat,
        out_type=jax.ShapeDtypeStruct((NC, SEGS, D), jnp.float32),
        mesh=mesh,
        scratch_types=[
            pltpu.VMEM((BLK, D), jnp.float32) for _ in range(NBUF)
        ] + [pltpu.VMEM((BLK,), jnp.int32) for _ in range(NBUF)] + [
            pltpu.VMEM_SHARED((SEGS, D), jnp.float32),
            pltpu.SemaphoreType.DMA,
            pltpu.SemaphoreType.DMA,
            pltpu.SemaphoreType.DMA,
            pltpu.SemaphoreType.DMA,
        ],
        compiler_params=pltpu.CompilerParams(use_tc_tiling_on_sc=False),
    )
    sc_partials = sc_kernel(feats, ids)

    ids_tc = ids.reshape(ROWS // TCR, 1, TCR)
    tc_partial = pl.pallas_call(
        _tc_body,
        out_shape=jax.ShapeDtypeStruct((SEGS_PAD, D), jnp.float32),
        grid=(TC_NBLK,),
        in_specs=[
            pl.BlockSpec((TCR, D), lambda i: (SC_ROWS // TCR + i, 0)),
            pl.BlockSpec((1, 1, TCR), lambda i: (SC_ROWS // TCR + i, 0, 0)),
        ],
        out_specs=pl.BlockSpec((SEGS_PAD, D), lambda i: (0, 0)),
    )(feats, ids_tc)

    grid = 10
    seg_blk = SEGS // grid
    return pl.pallas_call(
        _combine_body,
        out_shape=jax.ShapeDtypeStruct((SEGS, D), jnp.float32),
        grid=(grid,),
        in_specs=[
            pl.BlockSpec((NC, seg_blk, D), lambda i: (0, i, 0)),
            pl.BlockSpec((seg_blk, D), lambda i: (i, 0)),
        ],
        out_specs=pl.BlockSpec((seg_blk, D), lambda i: (i, 0)),
    )(sc_partials, tc_partial)


def kernel(feats, segment_ids, num_segments):
    return _run(feats, segment_ids, num_segments)


# speedup vs baseline: 1.7088x; 1.7088x over previous
import jax
import jax.numpy as jnp
from jax import lax
from jax.experimental import pallas as pl
from jax.experimental.pallas import tpu as pltpu
from jax.experimental.pallas import tpu_sc as plsc

NC = 2
NS = 16
LANES = 16
NBUF = 5

ROWS = 320000
D = 128
SEGS = 10000
DC = D // NC
BLK = 128
SUP = 2
SUP_ROWS = BLK * SUP
NSUP = ROWS // SUP_ROWS
KPT = (NSUP + NS - 1) // NS

ZBLK = 512
NZ = (SEGS + ZBLK - 1) // ZBLK


def _body(feats_hbm, ids_hbm, out_hbm, bufs, idxs, acc, sem_l, sem_s):
    c = lax.axis_index("c")
    s = lax.axis_index("s")

    def fire_load(k, slot):
        r0 = (s + k * NS) * SUP_ROWS
        pltpu.async_copy(
            feats_hbm.at[pl.ds(r0, SUP_ROWS), pl.ds(c * DC, DC)],
            bufs[slot], sem_l[slot])
        for j in range(SUP):
            pltpu.async_copy(ids_hbm.at[pl.ds(r0 + j * BLK, BLK)],
                             idxs[slot][j], sem_l[slot])

    def drain_load(slot):
        pltpu.make_async_copy(
            feats_hbm.at[pl.ds(0, SUP_ROWS), pl.ds(c * DC, DC)],
            bufs[slot], sem_l[slot]).wait()
        for j in range(SUP):
            pltpu.make_async_copy(ids_hbm.at[pl.ds(0, BLK)],
                                  idxs[slot][j], sem_l[slot]).wait()

    def fire_scatter(slot):
        for j in range(SUP):
            pltpu.async_copy(bufs[slot].at[pl.ds(j * BLK, BLK), :],
                             acc.at[idxs[slot][j]], sem_s, add=True)

    def drain_scatter(slot):
        for j in range(SUP):
            pltpu.make_async_copy(bufs[slot].at[pl.ds(j * BLK, BLK), :],
                                  acc.at[idxs[slot][j]], sem_s).wait()

    def valid(k):
        return (s + k * NS) < NSUP

    zeros16 = jnp.zeros((LANES,), jnp.float32)

    def zero_row(i, _):
        for t in range(DC // LANES):
            bufs[0][i, pl.ds(t * LANES, LANES)] = zeros16
        return 0

    lax.fori_loop(0, SUP_ROWS, zero_row, 0)

    for z in range((SEGS + SUP_ROWS - 1) // SUP_ROWS):
        nrows = min(SUP_ROWS, SEGS - z * SUP_ROWS)

        @pl.when(z % NS == s)
        def _():
            pltpu.sync_copy(bufs[0].at[pl.ds(0, nrows), :],
                            acc.at[pl.ds(z * SUP_ROWS, nrows), :])

    plsc.subcore_barrier()

    for p in range(NBUF - 1):
        @pl.when(valid(p))
        def _():
            fire_load(p, p)

    def step(it, _):
        for r in range(NBUF):
            k = NBUF * it + r

            @pl.when(valid(k))
            def _():
                drain_load(r)
                fire_scatter(r)

                @pl.when(k >= 1)
                def _():
                    drain_scatter((r + NBUF - 1) % NBUF)

                @pl.when(valid(k + NBUF - 1))
                def _():
                    fire_load(k + NBUF - 1, (r + NBUF - 1) % NBUF)

        return 0

    lax.fori_loop(0, (KPT + NBUF - 1) // NBUF, step, 0)

    drain_scatter(0)

    plsc.subcore_barrier()

    NZD = (SEGS + ZBLK - 1) // ZBLK
    for z in range(NZD):
        nrows = min(ZBLK, SEGS - z * ZBLK)

        @pl.when(z % NS == s)
        def _():
            pltpu.sync_copy(
                acc.at[pl.ds(z * ZBLK, nrows), :],
                out_hbm.at[pl.ds(z * ZBLK, nrows), pl.ds(c * DC, DC)])


def _body_flat(feats_hbm, ids_hbm, out_hbm,
               b0, b1, b2, b3, b4,
               i00, i01, i10, i11, i20, i21, i30, i31, i40, i41,
               acc, sl0, sl1, sl2, sl3, sl4, sem_s):
    _body(feats_hbm, ids_hbm, out_hbm,
          (b0, b1, b2, b3, b4),
          ((i00, i01), (i10, i11), (i20, i21), (i30, i31), (i40, i41)),
          acc, (sl0, sl1, sl2, sl3, sl4), sem_s)


@jax.jit
def _run(feats, segment_ids, num_segments):
    del num_segments
    ids = segment_ids.astype(jnp.int32)
    mesh = plsc.VectorSubcoreMesh(core_axis_name="c", subcore_axis_name="s")
    grid_kernel = pl.kernel(
        _body_flat,
        out_type=jax.ShapeDtypeStruct((SEGS, D), jnp.float32),
        mesh=mesh,
        scratch_types=[
            pltpu.VMEM((SUP_ROWS, DC), jnp.float32) for _ in range(NBUF)
        ] + [pltpu.VMEM((BLK,), jnp.int32) for _ in range(NBUF * SUP)] + [
            pltpu.VMEM_SHARED((SEGS, DC), jnp.float32),
        ] + [pltpu.SemaphoreType.DMA for _ in range(NBUF + 1)],
        compiler_params=pltpu.CompilerParams(use_tc_tiling_on_sc=False),
    )
    return grid_kernel(feats, ids)


def kernel(feats, segment_ids, num_segments):
    return _run(feats, segment_ids, num_segments)
